# baseline (speedup 1.0000x reference)
import jax
import jax.numpy as jnp
import numpy as np
from jax.experimental import pallas as pl
from jax.experimental.pallas import tpu as pltpu

H, W, PART = 16, 16, 4
EPS = 1e-5
HW = H * W
PR = H // PART
HALF = PR // 2
GSUB = 4
P, G, C = 48, 48, 64


def _band_mask():
    ai, bi = np.indices((HW, HW))
    lo = np.clip((bi // W) - HALF, 0, H - PR) * W
    m = np.where((ai >= lo) & (ai < lo + PR * W), 0.0, -np.inf)
    return m


_MASK = _band_mask()


def _fused_kernel(gal_ref, pf_ref, fcw_ref, m_ref, mt_ref, scal_ref,
                  out_ref, gft_ref, stats_ref):
    i = pl.program_id(0)

    @pl.when(i == 0)
    def _transpose_gallery():
        gft_ref[...] = jnp.transpose(
            gal_ref[...], (0, 2, 1)).astype(jnp.bfloat16).reshape(
                G * HW, C)

    pf0 = pf_ref[0].astype(jnp.bfloat16)
    mask = m_ref[...][None]
    maskt = mt_ref[...][None]
    s1_parts, s2_parts = [], []
    for j in range(G // GSUB):
        sc = jnp.dot(gft_ref[j * GSUB * HW:(j + 1) * GSUB * HW, :], pf0,
                     preferred_element_type=jnp.float32)
        sc3 = sc.astype(jnp.bfloat16).reshape(GSUB, HW, HW)
        s1_parts.append(jnp.max(sc3 + mask, axis=1))
        s2_parts.append(jnp.max(sc3 + maskt, axis=2))
    s1 = jnp.concatenate(s1_parts, axis=0).astype(jnp.float32)
    s2 = jnp.concatenate(s2_parts, axis=0).astype(jnp.float32)

    t = s1 + s2
    fcw = fcw_ref[...]
    w = jnp.sum(t * fcw, axis=1, keepdims=True)
    sv = jnp.sum(t, axis=1, keepdims=True)
    sq = jnp.sum(s1 * s1 + s2 * s2, axis=1, keepdims=True)
    stats_ref[i] = jnp.concatenate([w, sv, sq], axis=1)

    @pl.when(i == P - 1)
    def _epilogue():
        stats = stats_ref[...]
        w_raw = stats[:, :, 0]
        svs = stats[:, :, 1]
        sqs = stats[:, :, 2]
        bn_gamma = scal_ref[0, 0]
        bn_beta = scal_ref[0, 1]
        fc_b = scal_ref[0, 2]
        lg = scal_ref[0, 3]
        lb = scal_ref[0, 4]

        n1 = jnp.float32(2 * P * G * HW)
        m = jnp.sum(svs) / n1
        v = jnp.sum(sqs) / n1 - m * m
        a = bn_gamma * jax.lax.rsqrt(v + EPS)
        s_w = jnp.sum(fcw_ref[...])
        z = a * (w_raw - 2.0 * m * s_w) + 2.0 * (bn_beta * s_w + fc_b)

        npairs = jnp.float32(P * G)
        mz = jnp.sum(z) / npairs
        d = z - mz
        vz = jnp.sum(d * d) / npairs
        zn = lg * d * jax.lax.rsqrt(vz + EPS) + lb
        out_ref[...] = jax.nn.sigmoid(zn)


def kernel(prob_fea, gal_fea, bn_gamma, bn_beta, fc_w, fc_b, lbn_gamma,
           lbn_beta):
    p, c = prob_fea.shape[0], prob_fea.shape[1]
    g = gal_fea.shape[0]
    pf = prob_fea.reshape(p, c, HW)
    gal = gal_fea.reshape(g, c, HW)
    fcw = fc_w.reshape(1, HW)
    m = jnp.asarray(_MASK.astype(np.float32)).astype(jnp.bfloat16)
    mt = jnp.asarray(_MASK.T.copy().astype(np.float32)).astype(
        jnp.bfloat16)
    scal = jnp.concatenate(
        [bn_gamma, bn_beta, fc_b, lbn_gamma, lbn_beta]).reshape(1, 5)

    out = pl.pallas_call(
        _fused_kernel,
        grid=(p,),
        in_specs=[
            pl.BlockSpec((g, c, HW), lambda i: (0, 0, 0)),
            pl.BlockSpec((1, c, HW), lambda i: (i, 0, 0)),
            pl.BlockSpec((1, HW), lambda i: (0, 0)),
            pl.BlockSpec((HW, HW), lambda i: (0, 0)),
            pl.BlockSpec((HW, HW), lambda i: (0, 0)),
            pl.BlockSpec((1, 5), lambda i: (0, 0)),
        ],
        out_specs=pl.BlockSpec((p, g), lambda i: (0, 0)),
        out_shape=jax.ShapeDtypeStruct((p, g), jnp.float32),
        scratch_shapes=[
            pltpu.VMEM((g * HW, c), jnp.bfloat16),
            pltpu.VMEM((p, g, 3), jnp.float32),
        ],
        compiler_params=pltpu.CompilerParams(
            dimension_semantics=("arbitrary",),
        ),
    )(gal, pf, fcw, m, mt, scal)
    return out


# speedup vs baseline: 1.0162x; 1.0162x over previous
import jax
import jax.numpy as jnp
import numpy as np
from jax.experimental import pallas as pl
from jax.experimental.pallas import tpu as pltpu

H, W, PART = 16, 16, 4
EPS = 1e-5
HW = H * W
PR = H // PART
HALF = PR // 2
GSUB = 16
P, G, C = 48, 48, 64


def _band_mask():
    ai, bi = np.indices((HW, HW))
    lo = np.clip((bi // W) - HALF, 0, H - PR) * W
    m = np.where((ai >= lo) & (ai < lo + PR * W), 0.0, -np.inf)
    return m


_MASK = _band_mask()


def _fused_kernel(gal_ref, pf_ref, fcw_ref, m_ref, mt_ref, scal_ref,
                  out_ref, gft_ref, stats_ref):
    i = pl.program_id(0)

    @pl.when(i == 0)
    def _transpose_gallery():
        gft_ref[...] = jnp.transpose(
            gal_ref[...], (0, 2, 1)).astype(jnp.bfloat16).reshape(
                G * HW, C)

    pf0 = pf_ref[0].astype(jnp.bfloat16)
    mask = m_ref[...][None]
    maskt = mt_ref[...][None]
    s1_parts, s2_parts = [], []
    for j in range(G // GSUB):
        sc = jnp.dot(gft_ref[j * GSUB * HW:(j + 1) * GSUB * HW, :], pf0,
                     preferred_element_type=jnp.float32)
        sc3 = sc.astype(jnp.bfloat16).reshape(GSUB, HW, HW)
        s1_parts.append(jnp.max(sc3 + mask, axis=1))
        sct3 = jnp.transpose(sc3, (0, 2, 1))
        s2_parts.append(jnp.max(sct3 + mask, axis=1))
    s1 = jnp.concatenate(s1_parts, axis=0).astype(jnp.float32)
    s2 = jnp.concatenate(s2_parts, axis=0).astype(jnp.float32)

    t = s1 + s2
    fcw = fcw_ref[...]
    w = jnp.sum(t * fcw, axis=1, keepdims=True)
    sv = jnp.sum(t, axis=1, keepdims=True)
    sq = jnp.sum(s1 * s1 + s2 * s2, axis=1, keepdims=True)
    stats_ref[i] = jnp.concatenate([w, sv, sq], axis=1)

    @pl.when(i == P - 1)
    def _epilogue():
        stats = stats_ref[...]
        w_raw = stats[:, :, 0]
        svs = stats[:, :, 1]
        sqs = stats[:, :, 2]
        bn_gamma = scal_ref[0, 0]
        bn_beta = scal_ref[0, 1]
        fc_b = scal_ref[0, 2]
        lg = scal_ref[0, 3]
        lb = scal_ref[0, 4]

        n1 = jnp.float32(2 * P * G * HW)
        m = jnp.sum(svs) / n1
        v = jnp.sum(sqs) / n1 - m * m
        a = bn_gamma * jax.lax.rsqrt(v + EPS)
        s_w = jnp.sum(fcw_ref[...])
        z = a * (w_raw - 2.0 * m * s_w) + 2.0 * (bn_beta * s_w + fc_b)

        npairs = jnp.float32(P * G)
        mz = jnp.sum(z) / npairs
        d = z - mz
        vz = jnp.sum(d * d) / npairs
        zn = lg * d * jax.lax.rsqrt(vz + EPS) + lb
        out_ref[...] = jax.nn.sigmoid(zn)


def kernel(prob_fea, gal_fea, bn_gamma, bn_beta, fc_w, fc_b, lbn_gamma,
           lbn_beta):
    p, c = prob_fea.shape[0], prob_fea.shape[1]
    g = gal_fea.shape[0]
    pf = prob_fea.reshape(p, c, HW)
    gal = gal_fea.reshape(g, c, HW)
    fcw = fc_w.reshape(1, HW)
    m = jnp.asarray(_MASK.astype(np.float32)).astype(jnp.bfloat16)
    mt = jnp.asarray(_MASK.T.copy().astype(np.float32)).astype(
        jnp.bfloat16)
    scal = jnp.concatenate(
        [bn_gamma, bn_beta, fc_b, lbn_gamma, lbn_beta]).reshape(1, 5)

    out = pl.pallas_call(
        _fused_kernel,
        grid=(p,),
        in_specs=[
            pl.BlockSpec((g, c, HW), lambda i: (0, 0, 0)),
            pl.BlockSpec((1, c, HW), lambda i: (i, 0, 0)),
            pl.BlockSpec((1, HW), lambda i: (0, 0)),
            pl.BlockSpec((HW, HW), lambda i: (0, 0)),
            pl.BlockSpec((HW, HW), lambda i: (0, 0)),
            pl.BlockSpec((1, 5), lambda i: (0, 0)),
        ],
        out_specs=pl.BlockSpec((p, g), lambda i: (0, 0)),
        out_shape=jax.ShapeDtypeStruct((p, g), jnp.float32),
        scratch_shapes=[
            pltpu.VMEM((g * HW, c), jnp.bfloat16),
            pltpu.VMEM((p, g, 3), jnp.float32),
        ],
        compiler_params=pltpu.CompilerParams(
            dimension_semantics=("arbitrary",),
        ),
    )(gal, pf, fcw, m, mt, scal)
    return out


# speedup vs baseline: 1.0165x; 1.0004x over previous
import jax
import jax.numpy as jnp
import numpy as np
from jax.experimental import pallas as pl
from jax.experimental.pallas import tpu as pltpu

H, W, PART = 16, 16, 4
EPS = 1e-5
HW = H * W
PR = H // PART
HALF = PR // 2
GSUB = 16
P, G, C = 48, 48, 64


def _band_mask():
    ai, bi = np.indices((HW, HW))
    lo = np.clip((bi // W) - HALF, 0, H - PR) * W
    m = np.where((ai >= lo) & (ai < lo + PR * W), 0.0, -np.inf)
    return m


_MASK = _band_mask()


def _fused_kernel(gal_ref, pf_ref, fcw_ref, m_ref, scal_ref,
                  out_ref, gft_ref, stats_ref):
    i = pl.program_id(0)

    @pl.when(i == 0)
    def _transpose_gallery():
        gft_ref[...] = jnp.transpose(
            gal_ref[...], (0, 2, 1)).astype(jnp.bfloat16).reshape(
                G * HW, C)

    pf0 = pf_ref[0].astype(jnp.bfloat16)
    mask = m_ref[...][None]
    s1_parts, s2_parts = [], []
    for j in range(G // GSUB):
        sc = jnp.dot(gft_ref[j * GSUB * HW:(j + 1) * GSUB * HW, :], pf0,
                     preferred_element_type=jnp.float32)
        sc3 = sc.astype(jnp.bfloat16).reshape(GSUB, HW, HW)
        s1_parts.append(jnp.max(sc3 + mask, axis=1))
        sct3 = jnp.transpose(sc3, (0, 2, 1))
        s2_parts.append(jnp.max(sct3 + mask, axis=1))
    s1 = jnp.concatenate(s1_parts, axis=0).astype(jnp.float32)
    s2 = jnp.concatenate(s2_parts, axis=0).astype(jnp.float32)

    t = s1 + s2
    fcw = fcw_ref[...]
    w = jnp.sum(t * fcw, axis=1, keepdims=True)
    sv = jnp.sum(t, axis=1, keepdims=True)
    sq = jnp.sum(s1 * s1 + s2 * s2, axis=1, keepdims=True)
    stats_ref[i] = jnp.concatenate([w, sv, sq], axis=1)

    @pl.when(i == P - 1)
    def _epilogue():
        stats = stats_ref[...]
        w_raw = stats[:, :, 0]
        svs = stats[:, :, 1]
        sqs = stats[:, :, 2]
        bn_gamma = scal_ref[0, 0]
        bn_beta = scal_ref[0, 1]
        fc_b = scal_ref[0, 2]
        lg = scal_ref[0, 3]
        lb = scal_ref[0, 4]

        n1 = jnp.float32(2 * P * G * HW)
        m = jnp.sum(svs) / n1
        v = jnp.sum(sqs) / n1 - m * m
        a = bn_gamma * jax.lax.rsqrt(v + EPS)
        s_w = jnp.sum(fcw_ref[...])
        z = a * (w_raw - 2.0 * m * s_w) + 2.0 * (bn_beta * s_w + fc_b)

        npairs = jnp.float32(P * G)
        mz = jnp.sum(z) / npairs
        d = z - mz
        vz = jnp.sum(d * d) / npairs
        zn = lg * d * jax.lax.rsqrt(vz + EPS) + lb
        out_ref[...] = jax.nn.sigmoid(zn)


def kernel(prob_fea, gal_fea, bn_gamma, bn_beta, fc_w, fc_b, lbn_gamma,
           lbn_beta):
    p, c = prob_fea.shape[0], prob_fea.shape[1]
    g = gal_fea.shape[0]
    pf = prob_fea.reshape(p, c, HW)
    gal = gal_fea.reshape(g, c, HW)
    fcw = fc_w.reshape(1, HW)
    m = jnp.asarray(_MASK.astype(np.float32)).astype(jnp.bfloat16)
    scal = jnp.concatenate(
        [bn_gamma, bn_beta, fc_b, lbn_gamma, lbn_beta]).reshape(1, 5)

    out = pl.pallas_call(
        _fused_kernel,
        grid=(p,),
        in_specs=[
            pl.BlockSpec((g, c, HW), lambda i: (0, 0, 0)),
            pl.BlockSpec((1, c, HW), lambda i: (i, 0, 0)),
            pl.BlockSpec((1, HW), lambda i: (0, 0)),
            pl.BlockSpec((HW, HW), lambda i: (0, 0)),
            pl.BlockSpec((1, 5), lambda i: (0, 0)),
        ],
        out_specs=pl.BlockSpec((p, g), lambda i: (0, 0)),
        out_shape=jax.ShapeDtypeStruct((p, g), jnp.float32),
        scratch_shapes=[
            pltpu.VMEM((g * HW, c), jnp.bfloat16),
            pltpu.VMEM((p, g, 3), jnp.float32),
        ],
        compiler_params=pltpu.CompilerParams(
            dimension_semantics=("arbitrary",),
        ),
    )(gal, pf, fcw, m, scal)
    return out
